# baseline (speedup 1.0000x reference)
import jax
import jax.numpy as jnp
from jax import lax
from jax.experimental import pallas as pl
from jax.experimental.pallas import tpu as pltpu

E = 64
H = 1024
I = 768
NT = 2048
RT = 256
TM = 128
NB = 4


def _body(x_hbm, gate_ref, gup_hbm, dn_hbm, out_ref, lg_ref,
          xa_ref, ya_ref, lrank_ref, posv_ref, wtv_ref, cntv_ref, offv_ref,
          gidx_s, pos_s, wt_s, cnt_s, off_s, sem, gup_buf, dn_buf,
          gsem_a, gsem_b, dsem, x_ref, xsem):
    e = pl.program_id(0)

    def gup_copy_a(src_e, slot):
        return pltpu.make_async_copy(gup_hbm.at[src_e, :I],
                                     gup_buf.at[slot, :I], gsem_a.at[slot])

    def gup_copy_b(src_e, slot):
        return pltpu.make_async_copy(gup_hbm.at[src_e, I:],
                                     gup_buf.at[slot, I:], gsem_b.at[slot])

    def dn_copy(src_e, slot):
        return pltpu.make_async_copy(dn_hbm.at[src_e], dn_buf.at[slot],
                                     dsem.at[slot])

    @pl.when(e == 0)
    def _prefetch():
        for k in range(NB):
            gup_copy_a(k, k).start()
            gup_copy_b(k, k).start()
            dn_copy(k, k).start()
        pltpu.make_async_copy(x_hbm, x_ref, xsem).start()

    @pl.when(e == 0)
    def _prologue():
        pltpu.make_async_copy(x_hbm, x_ref, xsem).wait()
        l = lax.dot_general(x_ref[:, :], gate_ref[:, :],
                            (((1,), (1,)), ((), ())),
                            preferred_element_type=jnp.float32)
        lg_ref[:, :] = l
        m = jnp.max(l, axis=1, keepdims=True)
        s = jnp.sum(jnp.exp(l - m), axis=1)
        w = 1.0 / s
        eid = jnp.argmax(l, axis=1).astype(jnp.int32)

        iota_e = lax.broadcasted_iota(jnp.int32, (RT, E), 1)
        tril = (lax.broadcasted_iota(jnp.int32, (RT, RT), 0) >
                lax.broadcasted_iota(jnp.int32, (RT, RT), 1)).astype(jnp.float32)
        carry = jnp.zeros((1, E), jnp.float32)
        for t in range(NT // RT):
            eid_t = eid[t * RT:(t + 1) * RT]
            oh = (eid_t[:, None] == iota_e).astype(jnp.float32)
            ranks = lax.dot_general(tril, oh, (((1,), (0,)), ((), ())),
                                    preferred_element_type=jnp.float32)
            lrank_ref[0, t * RT:(t + 1) * RT] = (
                jnp.sum(oh * ranks, axis=1) + jnp.sum(oh * carry, axis=1))
            carry = carry + jnp.sum(oh, axis=0, keepdims=True)
        triu = (lax.broadcasted_iota(jnp.int32, (E, E), 0) <
                lax.broadcasted_iota(jnp.int32, (E, E), 1)).astype(jnp.float32)
        offs = lax.dot_general(carry, triu, (((1,), (0,)), ((), ())),
                               preferred_element_type=jnp.float32)
        cntv_ref[0, :] = carry[0].astype(jnp.int32)
        offv_ref[0, :] = offs[0].astype(jnp.int32)
        oh_full = (eid[:, None] ==
                   lax.broadcasted_iota(jnp.int32, (NT, E), 1)).astype(jnp.float32)
        off_tok = jnp.sum(oh_full * offs, axis=1)
        posv_ref[0, :] = (lrank_ref[0, :] + off_tok).astype(jnp.int32)
        wtv_ref[0, :] = w

        for src, dst in ((posv_ref, pos_s), (wtv_ref, wt_s),
                         (cntv_ref, cnt_s), (offv_ref, off_s)):
            cp = pltpu.make_async_copy(src, dst, sem)
            cp.start()
            cp.wait()

        def inv(t, _):
            gidx_s[0, pos_s[0, t]] = t
            return 0

        lax.fori_loop(0, NT, inv, 0)

    slot = lax.rem(e, NB)
    gup_copy_a(e, slot).wait()
    gup_copy_b(e, slot).wait()
    dn_copy(e, slot).wait()

    start = off_s[0, e]
    cnt_e = cnt_s[0, e]
    n_tiles = (cnt_e + TM - 1) // TM

    def tile_body(j, _):
        base = start + j * TM
        rows = jnp.minimum(cnt_e - j * TM, TM)

        def gather(r, _):
            src = gidx_s[0, base + r]
            xa_ref[pl.ds(r, 1), :] = x_ref[pl.ds(src, 1), :]
            return 0

        lax.fori_loop(0, rows, gather, 0)
        h = lax.dot_general(xa_ref[:, :], gup_buf[slot],
                            (((1,), (1,)), ((), ())),
                            preferred_element_type=jnp.float32)
        hg = h[:, :I]
        hu = h[:, I:]
        inter = hg * jax.nn.sigmoid(hg) * hu
        ya_ref[:, :] = lax.dot_general(inter, dn_buf[slot],
                                       (((1,), (1,)), ((), ())),
                                       preferred_element_type=jnp.float32)

        def scatter(r, _):
            dst = gidx_s[0, base + r]
            out_ref[pl.ds(dst, 1), :] = ya_ref[pl.ds(r, 1), :] * wt_s[0, dst]
            return 0

        lax.fori_loop(0, rows, scatter, 0)
        return 0

    lax.fori_loop(0, n_tiles, tile_body, 0)

    @pl.when(e + NB < E)
    def _refill():
        gup_copy_a(e + NB, slot).start()
        gup_copy_b(e + NB, slot).start()
        dn_copy(e + NB, slot).start()


def kernel(hidden_states, gate, gate_up_proj, down_proj):
    bsz, seq, hd = hidden_states.shape
    x = hidden_states.reshape(NT, H)

    out, logits = pl.pallas_call(
        _body,
        grid=(E,),
        in_specs=[
            pl.BlockSpec(memory_space=pl.ANY),
            pl.BlockSpec((E, H), lambda e: (0, 0)),
            pl.BlockSpec(memory_space=pl.ANY),
            pl.BlockSpec(memory_space=pl.ANY),
        ],
        out_specs=[
            pl.BlockSpec((NT, H), lambda e: (0, 0)),
            pl.BlockSpec((NT, E), lambda e: (0, 0)),
        ],
        out_shape=[
            jax.ShapeDtypeStruct((NT, H), jnp.float32),
            jax.ShapeDtypeStruct((NT, E), jnp.float32),
        ],
        scratch_shapes=[
            pltpu.VMEM((TM, H), jnp.float32),
            pltpu.VMEM((TM, H), jnp.float32),
            pltpu.VMEM((1, NT), jnp.float32),
            pltpu.VMEM((1, NT), jnp.int32),
            pltpu.VMEM((1, NT), jnp.float32),
            pltpu.VMEM((1, E), jnp.int32),
            pltpu.VMEM((1, E), jnp.int32),
            pltpu.SMEM((1, NT), jnp.int32),
            pltpu.SMEM((1, NT), jnp.int32),
            pltpu.SMEM((1, NT), jnp.float32),
            pltpu.SMEM((1, E), jnp.int32),
            pltpu.SMEM((1, E), jnp.int32),
            pltpu.SemaphoreType.DMA,
            pltpu.VMEM((NB, 2 * I, H), jnp.float32),
            pltpu.VMEM((NB, H, I), jnp.float32),
            pltpu.SemaphoreType.DMA((NB,)),
            pltpu.SemaphoreType.DMA((NB,)),
            pltpu.SemaphoreType.DMA((NB,)),
            pltpu.VMEM((NT, H), jnp.float32),
            pltpu.SemaphoreType.DMA,
        ],
        compiler_params=pltpu.CompilerParams(
            dimension_semantics=("arbitrary",)),
    )(x, gate, gate_up_proj, down_proj)

    return out.reshape(bsz, seq, hd), logits


# speedup vs baseline: 1.0153x; 1.0153x over previous
import jax
import jax.numpy as jnp
from jax import lax
from jax.experimental import pallas as pl
from jax.experimental.pallas import tpu as pltpu

E = 64
H = 1024
I = 768
NT = 2048
RT = 256
TM = 128
NB = 4


def _body(x_hbm, gate_ref, gup_hbm, dn_hbm, out_ref, lg_ref,
          xa_ref, ya_ref, lrank_ref, posv_ref, wtv_ref, cntv_ref, offv_ref,
          gidx_s, pos_s, wt_s, cnt_s, off_s, sem, gup_buf, dn_buf, gsem, dsem,
          x_ref, xsem):
    e = pl.program_id(0)

    def gup_copy(src_e, slot):
        return pltpu.make_async_copy(gup_hbm.at[src_e], gup_buf.at[slot],
                                     gsem.at[slot])

    def dn_copy(src_e, slot):
        return pltpu.make_async_copy(dn_hbm.at[src_e], dn_buf.at[slot],
                                     dsem.at[slot])

    @pl.when(e == 0)
    def _prefetch():
        pltpu.make_async_copy(x_hbm, x_ref, xsem).start()
        for k in range(NB):
            gup_copy(k, k).start()
            dn_copy(k, k).start()

    @pl.when(e == 0)
    def _prologue():
        pltpu.make_async_copy(x_hbm, x_ref, xsem).wait()
        l = lax.dot_general(x_ref[:, :], gate_ref[:, :],
                            (((1,), (1,)), ((), ())),
                            preferred_element_type=jnp.float32)
        lg_ref[:, :] = l
        m = jnp.max(l, axis=1, keepdims=True)
        s = jnp.sum(jnp.exp(l - m), axis=1)
        w = 1.0 / s
        eid = jnp.argmax(l, axis=1).astype(jnp.int32)

        iota_e = lax.broadcasted_iota(jnp.int32, (RT, E), 1)
        tril = (lax.broadcasted_iota(jnp.int32, (RT, RT), 0) >
                lax.broadcasted_iota(jnp.int32, (RT, RT), 1)).astype(jnp.float32)
        carry = jnp.zeros((1, E), jnp.float32)
        for t in range(NT // RT):
            eid_t = eid[t * RT:(t + 1) * RT]
            oh = (eid_t[:, None] == iota_e).astype(jnp.float32)
            ranks = lax.dot_general(tril, oh, (((1,), (0,)), ((), ())),
                                    preferred_element_type=jnp.float32)
            lrank_ref[0, t * RT:(t + 1) * RT] = (
                jnp.sum(oh * ranks, axis=1) + jnp.sum(oh * carry, axis=1))
            carry = carry + jnp.sum(oh, axis=0, keepdims=True)
        triu = (lax.broadcasted_iota(jnp.int32, (E, E), 0) <
                lax.broadcasted_iota(jnp.int32, (E, E), 1)).astype(jnp.float32)
        offs = lax.dot_general(carry, triu, (((1,), (0,)), ((), ())),
                               preferred_element_type=jnp.float32)
        cntv_ref[0, :] = carry[0].astype(jnp.int32)
        offv_ref[0, :] = offs[0].astype(jnp.int32)
        oh_full = (eid[:, None] ==
                   lax.broadcasted_iota(jnp.int32, (NT, E), 1)).astype(jnp.float32)
        off_tok = jnp.sum(oh_full * offs, axis=1)
        posv_ref[0, :] = (lrank_ref[0, :] + off_tok).astype(jnp.int32)
        wtv_ref[0, :] = w

        for src, dst in ((posv_ref, pos_s), (wtv_ref, wt_s),
                         (cntv_ref, cnt_s), (offv_ref, off_s)):
            cp = pltpu.make_async_copy(src, dst, sem)
            cp.start()
            cp.wait()

        def inv(t, _):
            gidx_s[0, pos_s[0, t]] = t
            return 0

        lax.fori_loop(0, NT, inv, 0)

    slot = lax.rem(e, NB)
    gup_copy(e, slot).wait()
    dn_copy(e, slot).wait()

    start = off_s[0, e]
    cnt_e = cnt_s[0, e]
    n_tiles = (cnt_e + TM - 1) // TM

    def tile_body(j, _):
        base = start + j * TM
        rows = jnp.minimum(cnt_e - j * TM, TM)

        def gather(r, _):
            src = gidx_s[0, base + r]
            xa_ref[pl.ds(r, 1), :] = x_ref[pl.ds(src, 1), :]
            return 0

        lax.fori_loop(0, rows, gather, 0)
        h = lax.dot_general(xa_ref[:, :], gup_buf[slot],
                            (((1,), (1,)), ((), ())),
                            preferred_element_type=jnp.float32)
        hg = h[:, :I]
        hu = h[:, I:]
        inter = hg * jax.nn.sigmoid(hg) * hu
        ya_ref[:, :] = lax.dot_general(inter, dn_buf[slot],
                                       (((1,), (1,)), ((), ())),
                                       preferred_element_type=jnp.float32)

        def scatter(r, _):
            dst = gidx_s[0, base + r]
            out_ref[pl.ds(dst, 1), :] = ya_ref[pl.ds(r, 1), :] * wt_s[0, dst]
            return 0

        lax.fori_loop(0, rows, scatter, 0)
        return 0

    lax.fori_loop(0, n_tiles, tile_body, 0)

    @pl.when(e + NB < E)
    def _refill():
        gup_copy(e + NB, slot).start()
        dn_copy(e + NB, slot).start()


def kernel(hidden_states, gate, gate_up_proj, down_proj):
    bsz, seq, hd = hidden_states.shape
    x = hidden_states.reshape(NT, H)

    out, logits = pl.pallas_call(
        _body,
        grid=(E,),
        in_specs=[
            pl.BlockSpec(memory_space=pl.ANY),
            pl.BlockSpec((E, H), lambda e: (0, 0)),
            pl.BlockSpec(memory_space=pl.ANY),
            pl.BlockSpec(memory_space=pl.ANY),
        ],
        out_specs=[
            pl.BlockSpec((NT, H), lambda e: (0, 0)),
            pl.BlockSpec((NT, E), lambda e: (0, 0)),
        ],
        out_shape=[
            jax.ShapeDtypeStruct((NT, H), jnp.float32),
            jax.ShapeDtypeStruct((NT, E), jnp.float32),
        ],
        scratch_shapes=[
            pltpu.VMEM((TM, H), jnp.float32),
            pltpu.VMEM((TM, H), jnp.float32),
            pltpu.VMEM((1, NT), jnp.float32),
            pltpu.VMEM((1, NT), jnp.int32),
            pltpu.VMEM((1, NT), jnp.float32),
            pltpu.VMEM((1, E), jnp.int32),
            pltpu.VMEM((1, E), jnp.int32),
            pltpu.SMEM((1, NT), jnp.int32),
            pltpu.SMEM((1, NT), jnp.int32),
            pltpu.SMEM((1, NT), jnp.float32),
            pltpu.SMEM((1, E), jnp.int32),
            pltpu.SMEM((1, E), jnp.int32),
            pltpu.SemaphoreType.DMA,
            pltpu.VMEM((NB, 2 * I, H), jnp.float32),
            pltpu.VMEM((NB, H, I), jnp.float32),
            pltpu.SemaphoreType.DMA((NB,)),
            pltpu.SemaphoreType.DMA((NB,)),
            pltpu.VMEM((NT, H), jnp.float32),
            pltpu.SemaphoreType.DMA,
        ],
        compiler_params=pltpu.CompilerParams(
            dimension_semantics=("arbitrary",)),
    )(x, gate, gate_up_proj, down_proj)

    return out.reshape(bsz, seq, hd), logits


# speedup vs baseline: 1.0277x; 1.0122x over previous
import functools

import jax
import jax.numpy as jnp
from jax import lax
from jax.experimental import pallas as pl
from jax.experimental.pallas import tpu as pltpu
from jax.experimental.pallas import tpu_sc as plsc

E = 64
H = 1024
I = 768
NT = 2048
RT = 256
TM = 128
NB = 4


def _router_body(x_ref, gate_ref, lg_ref, pos_ref, wt_ref, cnt_ref, off_ref,
                 lrank_ref):
    l = lax.dot_general(x_ref[:, :], gate_ref[:, :],
                        (((1,), (1,)), ((), ())),
                        preferred_element_type=jnp.float32)
    lg_ref[:, :] = l
    m = jnp.max(l, axis=1, keepdims=True)
    s = jnp.sum(jnp.exp(l - m), axis=1)
    eid = jnp.argmax(l, axis=1).astype(jnp.int32)

    iota_e = lax.broadcasted_iota(jnp.int32, (RT, E), 1)
    tril = (lax.broadcasted_iota(jnp.int32, (RT, RT), 0) >
            lax.broadcasted_iota(jnp.int32, (RT, RT), 1)).astype(jnp.float32)
    carry = jnp.zeros((1, E), jnp.float32)
    for t in range(NT // RT):
        eid_t = eid[t * RT:(t + 1) * RT]
        oh = (eid_t[:, None] == iota_e).astype(jnp.float32)
        ranks = lax.dot_general(tril, oh, (((1,), (0,)), ((), ())),
                                preferred_element_type=jnp.float32)
        lrank_ref[0, t * RT:(t + 1) * RT] = (
            jnp.sum(oh * ranks, axis=1) + jnp.sum(oh * carry, axis=1))
        carry = carry + jnp.sum(oh, axis=0, keepdims=True)
    triu = (lax.broadcasted_iota(jnp.int32, (E, E), 0) <
            lax.broadcasted_iota(jnp.int32, (E, E), 1)).astype(jnp.float32)
    offs = lax.dot_general(carry, triu, (((1,), (0,)), ((), ())),
                           preferred_element_type=jnp.float32)
    cnt_ref[0, :] = carry[0].astype(jnp.int32)
    off_ref[0, :] = offs[0].astype(jnp.int32)
    oh_full = (eid[:, None] ==
               lax.broadcasted_iota(jnp.int32, (NT, E), 1)).astype(jnp.float32)
    off_tok = jnp.sum(oh_full * offs, axis=1)
    pos_ref[0, :] = (lrank_ref[0, :] + off_tok).astype(jnp.int32)
    wt_ref[0, :] = 1.0 / s


def _gemm_body(gidx_p, wt_p, cnt_p, off_p,
               x_ref, gup_hbm, dn_hbm, out_ref,
               xa_ref, ya_ref, gup_buf, dn_buf, gsem, dsem):
    e = pl.program_id(0)

    def gup_copy(src_e, slot):
        return pltpu.make_async_copy(gup_hbm.at[src_e], gup_buf.at[slot],
                                     gsem.at[slot])

    def dn_copy(src_e, slot):
        return pltpu.make_async_copy(dn_hbm.at[src_e], dn_buf.at[slot],
                                     dsem.at[slot])

    @pl.when(e == 0)
    def _prefetch():
        for k in range(NB):
            gup_copy(k, k).start()
            dn_copy(k, k).start()

    slot = lax.rem(e, NB)
    gup_copy(e, slot).wait()
    dn_copy(e, slot).wait()

    start = off_p[e]
    cnt_e = cnt_p[e]
    n_tiles = (cnt_e + TM - 1) // TM

    def tile_body(j, _):
        base = start + j * TM
        rows = jnp.minimum(cnt_e - j * TM, TM)

        def gather(r, _):
            src = gidx_p[base + r]
            xa_ref[pl.ds(r, 1), :] = x_ref[pl.ds(src, 1), :]
            return 0

        lax.fori_loop(0, rows, gather, 0)
        h = lax.dot_general(xa_ref[:, :], gup_buf[slot],
                            (((1,), (1,)), ((), ())),
                            preferred_element_type=jnp.float32)
        hg = h[:, :I]
        hu = h[:, I:]
        inter = hg * jax.nn.sigmoid(hg) * hu
        ya_ref[:, :] = lax.dot_general(inter, dn_buf[slot],
                                       (((1,), (1,)), ((), ())),
                                       preferred_element_type=jnp.float32)

        def scatter(r, _):
            dst = gidx_p[base + r]
            out_ref[pl.ds(dst, 1), :] = ya_ref[pl.ds(r, 1), :] * wt_p[dst]
            return 0

        lax.fori_loop(0, rows, scatter, 0)
        return 0

    lax.fori_loop(0, n_tiles, tile_body, 0)

    @pl.when(e + NB < E)
    def _refill():
        gup_copy(e + NB, slot).start()
        dn_copy(e + NB, slot).start()


def _make_sc_inverse():
    mesh = plsc.VectorSubcoreMesh(core_axis_name="c", subcore_axis_name="s")

    @functools.partial(
        pl.kernel, mesh=mesh,
        out_type=jax.ShapeDtypeStruct((NT,), jnp.int32),
        compiler_params=pltpu.CompilerParams(needs_layout_passes=False),
        scratch_types=[
            pltpu.VMEM((NT,), jnp.int32),
            pltpu.VMEM((NT,), jnp.int32),
        ],
    )
    def inv_kernel(pos_hbm, out_hbm, pos_v, gidx_v):
        wid = lax.axis_index("s") * 2 + lax.axis_index("c")

        @pl.when(wid == 0)
        def _():
            pltpu.sync_copy(pos_hbm, pos_v)
            for c in range(NT // 16):
                idxs = pos_v[pl.ds(c * 16, 16)]
                vals = lax.iota(jnp.int32, 16) + (c * 16)
                plsc.store_scatter(gidx_v, [idxs], vals)
            pltpu.sync_copy(gidx_v, out_hbm)

    return inv_kernel


_sc_inverse = _make_sc_inverse()


def kernel(hidden_states, gate, gate_up_proj, down_proj):
    bsz, seq, hd = hidden_states.shape
    x = hidden_states.reshape(NT, H)

    logits, pos2, wt2, cnt2, off2 = pl.pallas_call(
        _router_body,
        grid=(1,),
        in_specs=[
            pl.BlockSpec((NT, H), lambda i: (0, 0)),
            pl.BlockSpec((E, H), lambda i: (0, 0)),
        ],
        out_specs=[
            pl.BlockSpec((NT, E), lambda i: (0, 0)),
            pl.BlockSpec((1, NT), lambda i: (0, 0)),
            pl.BlockSpec((1, NT), lambda i: (0, 0)),
            pl.BlockSpec((1, E), lambda i: (0, 0)),
            pl.BlockSpec((1, E), lambda i: (0, 0)),
        ],
        out_shape=[
            jax.ShapeDtypeStruct((NT, E), jnp.float32),
            jax.ShapeDtypeStruct((1, NT), jnp.int32),
            jax.ShapeDtypeStruct((1, NT), jnp.float32),
            jax.ShapeDtypeStruct((1, E), jnp.int32),
            jax.ShapeDtypeStruct((1, E), jnp.int32),
        ],
        scratch_shapes=[pltpu.VMEM((1, NT), jnp.float32)],
    )(x, gate)

    gidx = _sc_inverse(pos2.reshape(NT))

    grid_spec = pltpu.PrefetchScalarGridSpec(
        num_scalar_prefetch=4,
        grid=(E,),
        in_specs=[
            pl.BlockSpec((NT, H), lambda e, *_: (0, 0)),
            pl.BlockSpec(memory_space=pl.ANY),
            pl.BlockSpec(memory_space=pl.ANY),
        ],
        out_specs=pl.BlockSpec((NT, H), lambda e, *_: (0, 0)),
        scratch_shapes=[
            pltpu.VMEM((TM, H), jnp.float32),
            pltpu.VMEM((TM, H), jnp.float32),
            pltpu.VMEM((NB, 2 * I, H), jnp.float32),
            pltpu.VMEM((NB, H, I), jnp.float32),
            pltpu.SemaphoreType.DMA((NB,)),
            pltpu.SemaphoreType.DMA((NB,)),
        ],
    )
    out = pl.pallas_call(
        _gemm_body,
        grid_spec=grid_spec,
        out_shape=jax.ShapeDtypeStruct((NT, H), jnp.float32),
        compiler_params=pltpu.CompilerParams(
            dimension_semantics=("arbitrary",)),
    )(gidx, wt2.reshape(NT), cnt2.reshape(E), off2.reshape(E),
      x, gate_up_proj, down_proj)

    return out.reshape(bsz, seq, hd), logits
